# Initial kernel scaffold; baseline (speedup 1.0000x reference)
import jax
import jax.numpy as jnp
from jax.experimental import pallas as pl


def kernel(x, edge_index, Ws0, bs0, g0, be0, Wa0, ba0, Ws1, bs1, g1, be1, Wa1, ba1, Ws2, bs2, g2, be2, Wa2, ba2, Wskip, bskip):
    raise NotImplementedError("write your pallas kernel here")


# speedup vs baseline: 6.6212x; 6.6212x over previous
import functools

import jax
import jax.numpy as jnp
from jax import lax
from jax.experimental import pallas as pl
from jax.experimental.pallas import tpu as pltpu
from jax.experimental.pallas import tpu_sc as plsc

N = 10000
E = 320000
CH = 80
NTILES = 16
NPT = N // NTILES
ROWS_PER_TILE = (E // CH) // NTILES
BLK = 1000
GRID = N // BLK


def _prep_body(h_ref, wa_ref, ba_ref, hwa_ref, hwb_ref, ed_ref):
    h = h_ref[...]
    d = h.shape[1]
    w_half = hwa_ref.shape[1]
    a = jnp.dot(h, wa_ref[...], preferred_element_type=jnp.float32)
    a_dst = a[:, 0:1]
    a_src = a[:, 1:2]
    w = jnp.exp(a_src)
    ed_ref[...] = jnp.exp(a_dst + ba_ref[0, 0])
    hw = h * w
    pad = jnp.concatenate(
        [hw, w, jnp.zeros((h.shape[0], 2 * w_half - d - 1), jnp.float32)],
        axis=1)
    hwa_ref[...] = pad[:, :w_half]
    hwb_ref[...] = pad[:, w_half:]


def _make_prep(d, w_half):
    return pl.pallas_call(
        _prep_body,
        grid=(GRID,),
        in_specs=[
            pl.BlockSpec((BLK, d), lambda i: (i, 0)),
            pl.BlockSpec((d, 2), lambda i: (0, 0)),
            pl.BlockSpec((1, 1), lambda i: (0, 0)),
        ],
        out_specs=[
            pl.BlockSpec((BLK, w_half), lambda i: (i, 0)),
            pl.BlockSpec((BLK, w_half), lambda i: (i, 0)),
            pl.BlockSpec((BLK, 1), lambda i: (i, 0)),
        ],
        out_shape=[
            jax.ShapeDtypeStruct((N, w_half), jnp.float32),
            jax.ShapeDtypeStruct((N, w_half), jnp.float32),
            jax.ShapeDtypeStruct((N, 1), jnp.float32),
        ],
    )


def _make_sc(w_half):
    mesh = plsc.VectorSubcoreMesh(core_axis_name="c", subcore_axis_name="s")

    @functools.partial(
        pl.kernel,
        mesh=mesh,
        compiler_params=pltpu.CompilerParams(use_tc_tiling_on_sc=False),
        out_type=[
            jax.ShapeDtypeStruct((N, w_half), jnp.float32),
            jax.ShapeDtypeStruct((N, w_half), jnp.float32),
        ],
        scratch_types=[
            pltpu.VMEM((CH,), jnp.int32),
            pltpu.VMEM((CH,), jnp.int32),
            pltpu.VMEM((CH, w_half), jnp.float32),
            pltpu.VMEM_SHARED((N, w_half), jnp.float32),
            pltpu.SemaphoreType.DMA,
        ],
    )
    def sc_kernel(hwa, hwb, src_r, dst_r, zblk, ua, ub,
                  sidx, didx, rows, ush, sem):
        c = lax.axis_index("c")
        s = lax.axis_index("s")
        pltpu.sync_copy(zblk, ush.at[pl.ds(s * NPT, NPT)])
        plsc.subcore_barrier()

        base = s * ROWS_PER_TILE

        def run(hw_ref):
            def body_fn(k, carry):
                row = base + k
                pltpu.sync_copy(src_r.at[row], sidx)
                pltpu.sync_copy(dst_r.at[row], didx)
                pltpu.async_copy(hw_ref.at[sidx], rows, sem).wait()
                pltpu.sync_copy(rows, ush.at[didx], add=True)
                return carry
            lax.fori_loop(0, ROWS_PER_TILE, body_fn, 0)

        @pl.when(c == 0)
        def _():
            run(hwa)

        @pl.when(c == 1)
        def _():
            run(hwb)

        plsc.subcore_barrier()

        @pl.when(c == 0)
        def _():
            pltpu.sync_copy(ush.at[pl.ds(s * NPT, NPT)],
                            ua.at[pl.ds(s * NPT, NPT)])

        @pl.when(c == 1)
        def _():
            pltpu.sync_copy(ush.at[pl.ds(s * NPT, NPT)],
                            ub.at[pl.ds(s * NPT, NPT)])

    return sc_kernel


def _combine_body(ua_ref, ub_ref, ed_ref, h_ref, w1t_ref, w2t_ref,
                  bs_ref, g_ref, be_ref, out_ref):
    d = h_ref.shape[1]
    u = jnp.concatenate([ua_ref[...], ub_ref[...]], axis=1)
    t = u[:, :d]
    s_sum = u[:, d:d + 1]
    ed = ed_ref[...]
    r = ed / jnp.maximum(ed * s_sum, 1e-8)
    agg = t * r
    z = (jnp.dot(h_ref[...], w1t_ref[...], preferred_element_type=jnp.float32)
         + jnp.dot(agg, w2t_ref[...], preferred_element_type=jnp.float32)
         + bs_ref[...])
    mu = jnp.mean(z, axis=1, keepdims=True)
    var = jnp.mean((z - mu) ** 2, axis=1, keepdims=True)
    out_ref[...] = jnp.maximum(
        (z - mu) * lax.rsqrt(var + 1e-5) * g_ref[...] + be_ref[...], 0.0)


def _make_combine(d, w_half):
    return pl.pallas_call(
        _combine_body,
        grid=(GRID,),
        in_specs=[
            pl.BlockSpec((BLK, w_half), lambda i: (i, 0)),
            pl.BlockSpec((BLK, w_half), lambda i: (i, 0)),
            pl.BlockSpec((BLK, 1), lambda i: (i, 0)),
            pl.BlockSpec((BLK, d), lambda i: (i, 0)),
            pl.BlockSpec((d, 256), lambda i: (0, 0)),
            pl.BlockSpec((d, 256), lambda i: (0, 0)),
            pl.BlockSpec((1, 256), lambda i: (0, 0)),
            pl.BlockSpec((1, 256), lambda i: (0, 0)),
            pl.BlockSpec((1, 256), lambda i: (0, 0)),
        ],
        out_specs=pl.BlockSpec((BLK, 256), lambda i: (i, 0)),
        out_shape=jax.ShapeDtypeStruct((N, 256), jnp.float32),
    )


def _combine_final_body(ua_ref, ub_ref, ed_ref, h_ref, w1t_ref, w2t_ref,
                        bs_ref, g_ref, be_ref, x_ref, wskipt_ref,
                        bskip_ref, acc_ref):
    d = h_ref.shape[1]
    u = jnp.concatenate([ua_ref[...], ub_ref[...]], axis=1)
    t = u[:, :d]
    s_sum = u[:, d:d + 1]
    ed = ed_ref[...]
    r = ed / jnp.maximum(ed * s_sum, 1e-8)
    agg = t * r
    z = (jnp.dot(h_ref[...], w1t_ref[...], preferred_element_type=jnp.float32)
         + jnp.dot(agg, w2t_ref[...], preferred_element_type=jnp.float32)
         + bs_ref[...])
    mu = jnp.mean(z, axis=1, keepdims=True)
    var = jnp.mean((z - mu) ** 2, axis=1, keepdims=True)
    h3 = jnp.maximum(
        (z - mu) * lax.rsqrt(var + 1e-5) * g_ref[...] + be_ref[...], 0.0)
    xs = jnp.sum(x_ref[...], axis=0, keepdims=True)
    part = (jnp.sum(h3, axis=0, keepdims=True)
            + jnp.dot(xs, wskipt_ref[...], preferred_element_type=jnp.float32))

    @pl.when(pl.program_id(0) == 0)
    def _():
        acc_ref[...] = jnp.zeros_like(acc_ref)

    acc_ref[...] += part

    @pl.when(pl.program_id(0) == GRID - 1)
    def _():
        acc_ref[...] = acc_ref[...] * (1.0 / N) + bskip_ref[...]


def _make_combine_final(d, w_half, in_dim):
    return pl.pallas_call(
        _combine_final_body,
        grid=(GRID,),
        in_specs=[
            pl.BlockSpec((BLK, w_half), lambda i: (i, 0)),
            pl.BlockSpec((BLK, w_half), lambda i: (i, 0)),
            pl.BlockSpec((BLK, 1), lambda i: (i, 0)),
            pl.BlockSpec((BLK, d), lambda i: (i, 0)),
            pl.BlockSpec((d, 256), lambda i: (0, 0)),
            pl.BlockSpec((d, 256), lambda i: (0, 0)),
            pl.BlockSpec((1, 256), lambda i: (0, 0)),
            pl.BlockSpec((1, 256), lambda i: (0, 0)),
            pl.BlockSpec((1, 256), lambda i: (0, 0)),
            pl.BlockSpec((BLK, in_dim), lambda i: (i, 0)),
            pl.BlockSpec((in_dim, 256), lambda i: (0, 0)),
            pl.BlockSpec((1, 256), lambda i: (0, 0)),
        ],
        out_specs=pl.BlockSpec((1, 256), lambda i: (0, 0)),
        out_shape=jax.ShapeDtypeStruct((1, 256), jnp.float32),
    )


def kernel(x, edge_index, Ws0, bs0, g0, be0, Wa0, ba0, Ws1, bs1, g1, be1,
           Wa1, ba1, Ws2, bs2, g2, be2, Wa2, ba2, Wskip, bskip):
    src2d = edge_index[0].reshape(E // CH, CH)
    dst2d = edge_index[1].reshape(E // CH, CH)

    dims = (128, 256, 256)
    halves = (80, 144, 144)
    params = ((Ws0, bs0, g0, be0, Wa0, ba0),
              (Ws1, bs1, g1, be1, Wa1, ba1),
              (Ws2, bs2, g2, be2, Wa2, ba2))

    h = x
    for i in range(3):
        d = dims[i]
        w_half = halves[i]
        Ws, bs, g, be, Wa, ba = params[i]
        wa2 = jnp.stack([Wa[0, :d], Wa[0, d:]], axis=1)
        ba_arr = ba.reshape(1, 1)
        hwa, hwb, ed = _make_prep(d, w_half)(h, wa2, ba_arr)
        zblk = jnp.zeros((NPT, w_half), jnp.float32)
        ua, ub = _make_sc(w_half)(hwa, hwb, src2d, dst2d, zblk)
        w1t = Ws[:, :d].T
        w2t = Ws[:, d:].T
        if i < 2:
            h = _make_combine(d, w_half)(
                ua, ub, ed, h, w1t, w2t,
                bs.reshape(1, 256), g.reshape(1, 256), be.reshape(1, 256))
        else:
            acc = _make_combine_final(d, w_half, 128)(
                ua, ub, ed, h, w1t, w2t,
                bs.reshape(1, 256), g.reshape(1, 256), be.reshape(1, 256),
                x, Wskip.T, bskip.reshape(1, 256))
    return acc.reshape(256)


# speedup vs baseline: 7.1753x; 1.0837x over previous
import functools

import jax
import jax.numpy as jnp
from jax import lax
from jax.experimental import pallas as pl
from jax.experimental.pallas import tpu as pltpu
from jax.experimental.pallas import tpu_sc as plsc

N = 10000
E = 320000
CH = 128
NTILES = 16
NPT = N // NTILES
K_PT = 160
E_PAD = NTILES * K_PT * CH
BLK = 1000
GRID = N // BLK


def _prep_body(h_ref, wa_ref, ba_ref, hwa_ref, hwb_ref, ed_ref):
    h = h_ref[...]
    d = h.shape[1]
    w_half = hwa_ref.shape[1]
    a = jnp.dot(h, wa_ref[...], preferred_element_type=jnp.float32)
    a_dst = a[:, 0:1]
    a_src = a[:, 1:2]
    w = jnp.exp(a_src)
    ed_ref[...] = jnp.exp(a_dst + ba_ref[0, 0])
    hw = h * w
    pad = jnp.concatenate(
        [hw, w, jnp.zeros((h.shape[0], 2 * w_half - d - 1), jnp.float32)],
        axis=1)
    hwa_ref[...] = pad[:, :w_half]
    hwb_ref[...] = pad[:, w_half:]


def _make_prep(d, w_half):
    return pl.pallas_call(
        _prep_body,
        grid=(GRID,),
        in_specs=[
            pl.BlockSpec((BLK, d), lambda i: (i, 0)),
            pl.BlockSpec((d, 2), lambda i: (0, 0)),
            pl.BlockSpec((1, 1), lambda i: (0, 0)),
        ],
        out_specs=[
            pl.BlockSpec((BLK, w_half), lambda i: (i, 0)),
            pl.BlockSpec((BLK, w_half), lambda i: (i, 0)),
            pl.BlockSpec((BLK, 1), lambda i: (i, 0)),
        ],
        out_shape=[
            jax.ShapeDtypeStruct((N, w_half), jnp.float32),
            jax.ShapeDtypeStruct((N, w_half), jnp.float32),
            jax.ShapeDtypeStruct((N, 1), jnp.float32),
        ],
    )


def _make_sc(w_half):
    mesh = plsc.VectorSubcoreMesh(core_axis_name="c", subcore_axis_name="s")

    @functools.partial(
        pl.kernel,
        mesh=mesh,
        compiler_params=pltpu.CompilerParams(use_tc_tiling_on_sc=False),
        out_type=[
            jax.ShapeDtypeStruct((N, w_half), jnp.float32),
            jax.ShapeDtypeStruct((N, w_half), jnp.float32),
        ],
        scratch_types=[
            pltpu.VMEM((4, CH), jnp.int32),
            pltpu.VMEM((4, CH), jnp.int32),
            pltpu.VMEM((CH, w_half), jnp.float32),
            pltpu.VMEM((CH, w_half), jnp.float32),
            pltpu.VMEM_SHARED((N + 8, w_half), jnp.float32),
            pltpu.SemaphoreType.DMA,
            pltpu.SemaphoreType.DMA,
            pltpu.SemaphoreType.DMA,
            pltpu.SemaphoreType.DMA,
            pltpu.SemaphoreType.DMA,
            pltpu.SemaphoreType.DMA,
            pltpu.SemaphoreType.DMA,
            pltpu.SemaphoreType.DMA,
        ],
    )
    def sc_kernel(hwa, hwb, src_r, dst_r, zblk, zblk8, ua, ub,
                  sidx, didx, rows0, rows1, ush,
                  si0, si1, si2, si3, sg0, sg1, ss0, ss1):
        c = lax.axis_index("c")
        s = lax.axis_index("s")
        pltpu.sync_copy(zblk, ush.at[pl.ds(s * NPT, NPT)])

        @pl.when(s == NTILES - 1)
        def _():
            pltpu.sync_copy(zblk8, ush.at[pl.ds(N, 8)])

        plsc.subcore_barrier()

        rows = (rows0, rows1)
        si = (si0, si1, si2, si3)
        sg = (sg0, sg1)
        ss = (ss0, ss1)

        def fetch_idx(k, u):
            pltpu.async_copy(src_r.at[s, k], sidx.at[u], si[u])
            pltpu.async_copy(dst_r.at[s, k], didx.at[u], si[u])

        def wait_idx(k, u):
            pltpu.make_async_copy(src_r.at[s, k], sidx.at[u], si[u]).wait()
            pltpu.make_async_copy(dst_r.at[s, k], didx.at[u], si[u]).wait()

        def run(hw_ref):
            fetch_idx(0, 0)
            fetch_idx(1, 1)
            wait_idx(0, 0)
            pltpu.async_copy(hw_ref.at[sidx.at[0]], rows[0], sg[0])

            def body_fn(j, carry):
                for r in range(4):
                    k = 4 * j + r
                    b = r % 2
                    nb = 1 - b
                    pltpu.make_async_copy(
                        hw_ref.at[sidx.at[r]], rows[b], sg[b]).wait()

                    @pl.when(k >= 1)
                    def _():
                        pltpu.make_async_copy(
                            rows[nb], ush.at[didx.at[(r + 3) % 4]],
                            ss[nb]).wait()

                    @pl.when(k + 2 < K_PT)
                    def _():
                        fetch_idx(k + 2, (r + 2) % 4)

                    @pl.when(k + 1 < K_PT)
                    def _():
                        wait_idx(k + 1, (r + 1) % 4)
                        pltpu.async_copy(
                            hw_ref.at[sidx.at[(r + 1) % 4]], rows[nb], sg[nb])

                    pltpu.async_copy(
                        rows[b], ush.at[didx.at[r]], ss[b], add=True)
                return carry

            lax.fori_loop(0, K_PT // 4, body_fn, 0)
            pltpu.make_async_copy(
                rows[1], ush.at[didx.at[3]], ss[1]).wait()

        @pl.when(c == 0)
        def _():
            run(hwa)

        @pl.when(c == 1)
        def _():
            run(hwb)

        plsc.subcore_barrier()

        @pl.when(c == 0)
        def _():
            pltpu.sync_copy(ush.at[pl.ds(s * NPT, NPT)],
                            ua.at[pl.ds(s * NPT, NPT)])

        @pl.when(c == 1)
        def _():
            pltpu.sync_copy(ush.at[pl.ds(s * NPT, NPT)],
                            ub.at[pl.ds(s * NPT, NPT)])

    return sc_kernel


def _combine_body(ua_ref, ub_ref, ed_ref, h_ref, w1t_ref, w2t_ref,
                  bs_ref, g_ref, be_ref, out_ref):
    d = h_ref.shape[1]
    u = jnp.concatenate([ua_ref[...], ub_ref[...]], axis=1)
    t = u[:, :d]
    s_sum = u[:, d:d + 1]
    ed = ed_ref[...]
    r = ed / jnp.maximum(ed * s_sum, 1e-8)
    agg = t * r
    z = (jnp.dot(h_ref[...], w1t_ref[...], preferred_element_type=jnp.float32)
         + jnp.dot(agg, w2t_ref[...], preferred_element_type=jnp.float32)
         + bs_ref[...])
    mu = jnp.mean(z, axis=1, keepdims=True)
    var = jnp.mean((z - mu) ** 2, axis=1, keepdims=True)
    out_ref[...] = jnp.maximum(
        (z - mu) * lax.rsqrt(var + 1e-5) * g_ref[...] + be_ref[...], 0.0)


def _make_combine(d, w_half):
    return pl.pallas_call(
        _combine_body,
        grid=(GRID,),
        in_specs=[
            pl.BlockSpec((BLK, w_half), lambda i: (i, 0)),
            pl.BlockSpec((BLK, w_half), lambda i: (i, 0)),
            pl.BlockSpec((BLK, 1), lambda i: (i, 0)),
            pl.BlockSpec((BLK, d), lambda i: (i, 0)),
            pl.BlockSpec((d, 256), lambda i: (0, 0)),
            pl.BlockSpec((d, 256), lambda i: (0, 0)),
            pl.BlockSpec((1, 256), lambda i: (0, 0)),
            pl.BlockSpec((1, 256), lambda i: (0, 0)),
            pl.BlockSpec((1, 256), lambda i: (0, 0)),
        ],
        out_specs=pl.BlockSpec((BLK, 256), lambda i: (i, 0)),
        out_shape=jax.ShapeDtypeStruct((N, 256), jnp.float32),
    )


def _combine_final_body(ua_ref, ub_ref, ed_ref, h_ref, w1t_ref, w2t_ref,
                        bs_ref, g_ref, be_ref, x_ref, wskipt_ref,
                        bskip_ref, acc_ref):
    d = h_ref.shape[1]
    u = jnp.concatenate([ua_ref[...], ub_ref[...]], axis=1)
    t = u[:, :d]
    s_sum = u[:, d:d + 1]
    ed = ed_ref[...]
    r = ed / jnp.maximum(ed * s_sum, 1e-8)
    agg = t * r
    z = (jnp.dot(h_ref[...], w1t_ref[...], preferred_element_type=jnp.float32)
         + jnp.dot(agg, w2t_ref[...], preferred_element_type=jnp.float32)
         + bs_ref[...])
    mu = jnp.mean(z, axis=1, keepdims=True)
    var = jnp.mean((z - mu) ** 2, axis=1, keepdims=True)
    h3 = jnp.maximum(
        (z - mu) * lax.rsqrt(var + 1e-5) * g_ref[...] + be_ref[...], 0.0)
    xs = jnp.sum(x_ref[...], axis=0, keepdims=True)
    part = (jnp.sum(h3, axis=0, keepdims=True)
            + jnp.dot(xs, wskipt_ref[...], preferred_element_type=jnp.float32))

    @pl.when(pl.program_id(0) == 0)
    def _():
        acc_ref[...] = jnp.zeros_like(acc_ref)

    acc_ref[...] += part

    @pl.when(pl.program_id(0) == GRID - 1)
    def _():
        acc_ref[...] = acc_ref[...] * (1.0 / N) + bskip_ref[...]


def _make_combine_final(d, w_half, in_dim):
    return pl.pallas_call(
        _combine_final_body,
        grid=(GRID,),
        in_specs=[
            pl.BlockSpec((BLK, w_half), lambda i: (i, 0)),
            pl.BlockSpec((BLK, w_half), lambda i: (i, 0)),
            pl.BlockSpec((BLK, 1), lambda i: (i, 0)),
            pl.BlockSpec((BLK, d), lambda i: (i, 0)),
            pl.BlockSpec((d, 256), lambda i: (0, 0)),
            pl.BlockSpec((d, 256), lambda i: (0, 0)),
            pl.BlockSpec((1, 256), lambda i: (0, 0)),
            pl.BlockSpec((1, 256), lambda i: (0, 0)),
            pl.BlockSpec((1, 256), lambda i: (0, 0)),
            pl.BlockSpec((BLK, in_dim), lambda i: (i, 0)),
            pl.BlockSpec((in_dim, 256), lambda i: (0, 0)),
            pl.BlockSpec((1, 256), lambda i: (0, 0)),
        ],
        out_specs=pl.BlockSpec((1, 256), lambda i: (0, 0)),
        out_shape=jax.ShapeDtypeStruct((1, 256), jnp.float32),
    )


def kernel(x, edge_index, Ws0, bs0, g0, be0, Wa0, ba0, Ws1, bs1, g1, be1,
           Wa1, ba1, Ws2, bs2, g2, be2, Wa2, ba2, Wskip, bskip):
    pad = E_PAD - E
    src3 = jnp.concatenate(
        [edge_index[0], jnp.zeros((pad,), jnp.int32)]).reshape(
            NTILES, K_PT, CH)
    dst3 = jnp.concatenate(
        [edge_index[1], jnp.full((pad,), N, jnp.int32)]).reshape(
            NTILES, K_PT, CH)

    dims = (128, 256, 256)
    halves = (80, 144, 144)
    params = ((Ws0, bs0, g0, be0, Wa0, ba0),
              (Ws1, bs1, g1, be1, Wa1, ba1),
              (Ws2, bs2, g2, be2, Wa2, ba2))

    h = x
    for i in range(3):
        d = dims[i]
        w_half = halves[i]
        Ws, bs, g, be, Wa, ba = params[i]
        wa2 = jnp.stack([Wa[0, :d], Wa[0, d:]], axis=1)
        ba_arr = ba.reshape(1, 1)
        hwa, hwb, ed = _make_prep(d, w_half)(h, wa2, ba_arr)
        zblk = jnp.zeros((NPT, w_half), jnp.float32)
        zblk8 = jnp.zeros((8, w_half), jnp.float32)
        ua, ub = _make_sc(w_half)(hwa, hwb, src3, dst3, zblk, zblk8)
        w1t = Ws[:, :d].T
        w2t = Ws[:, d:].T
        if i < 2:
            h = _make_combine(d, w_half)(
                ua, ub, ed, h, w1t, w2t,
                bs.reshape(1, 256), g.reshape(1, 256), be.reshape(1, 256))
        else:
            acc = _make_combine_final(d, w_half, 128)(
                ua, ub, ed, h, w1t, w2t,
                bs.reshape(1, 256), g.reshape(1, 256), be.reshape(1, 256),
                x, Wskip.T, bskip.reshape(1, 256))
    return acc.reshape(256)
